# baseline (speedup 1.0000x reference)
import functools

import jax
import jax.numpy as jnp
from jax import lax
from jax.experimental import pallas as pl
from jax.experimental.pallas import tpu as pltpu
from jax.experimental.pallas import tpu_sc as plsc

WORDS = 1000000
FEATURES = 32
BATCH = 16384

NUM_CORES = 2
NUM_SUBCORES = 16
NUM_WORKERS = NUM_CORES * NUM_SUBCORES
B_PER_W = BATCH // NUM_WORKERS

PACK = 4
FAT = PACK * FEATURES
N_FAT = WORDS // PACK
CHUNK = 128
N_CHUNKS = B_PER_W // CHUNK
NBUF = 2
LANES = 16

_mesh = plsc.VectorSubcoreMesh(
    core_axis_name="c", subcore_axis_name="s",
    num_cores=NUM_CORES, num_subcores=NUM_SUBCORES)


@functools.partial(
    pl.kernel,
    out_type=jax.ShapeDtypeStruct((BATCH, FEATURES), jnp.float32),
    mesh=_mesh,
    scratch_types=[
        pltpu.VMEM((B_PER_W,), jnp.int32),
        pltpu.VMEM((B_PER_W,), jnp.int32),
        pltpu.VMEM((NBUF, CHUNK, FAT), jnp.float32),
        pltpu.VMEM((B_PER_W, FEATURES), jnp.float32),
        pltpu.SemaphoreType.DMA,
    ],
    compiler_params=pltpu.CompilerParams(needs_layout_passes=False),
)
def _gather_kernel(tab2, idx_hbm, out_hbm,
                   idx_v, fidx_v, fat_v, out_v, sem):
    wid = lax.axis_index("s") * NUM_CORES + lax.axis_index("c")
    base = wid * B_PER_W
    pltpu.sync_copy(idx_hbm.at[pl.ds(base, B_PER_W)], idx_v)

    def scale(m, carry):
        v = idx_v[pl.ds(m * LANES, LANES)]
        fidx_v[pl.ds(m * LANES, LANES)] = lax.shift_right_logical(v, 2)
        return carry

    lax.fori_loop(0, B_PER_W // LANES, scale, 0, unroll=False)

    def start_chunk(g, buf):
        pltpu.async_copy(
            tab2.at[fidx_v.at[pl.ds(g * CHUNK, CHUNK)]], fat_v.at[buf], sem)

    def wait_chunk():
        pltpu.make_async_copy(
            tab2.at[pl.ds(0, CHUNK)], fat_v.at[0], sem).wait()

    def select_chunk(g, buf):
        def rowgroup(m, carry):
            vrow = idx_v[pl.ds(g * CHUNK + m * LANES, LANES)]
            off = lax.bitwise_and(vrow, jnp.int32(PACK - 1)) * FEATURES
            i0 = m * LANES + lax.iota(jnp.int32, LANES)
            for col in range(FEATURES):
                vals = plsc.load_gather(fat_v.at[buf], [i0, off + col])
                plsc.store_scatter(
                    out_v,
                    [g * CHUNK + i0, jnp.full((LANES,), col, jnp.int32)],
                    vals)
            return carry

        lax.fori_loop(0, CHUNK // LANES, rowgroup, 0, unroll=False)

    start_chunk(0, 0)
    for g in range(N_CHUNKS):
        if g + 1 < N_CHUNKS:
            start_chunk(g + 1, (g + 1) % NBUF)
        wait_chunk()
        select_chunk(g, g % NBUF)

    pltpu.sync_copy(out_v, out_hbm.at[pl.ds(base, B_PER_W)])


def kernel(emb_vec, idx):
    tab2 = emb_vec.reshape(N_FAT, FAT)
    return _gather_kernel(tab2, idx.astype(jnp.int32))


# speedup vs baseline: 1.7111x; 1.7111x over previous
import functools

import jax
import jax.numpy as jnp
from jax import lax
from jax.experimental import pallas as pl
from jax.experimental.pallas import tpu as pltpu
from jax.experimental.pallas import tpu_sc as plsc

WORDS = 1000000
FEATURES = 32
BATCH = 16384

NUM_CORES = 2
NUM_SUBCORES = 16
NUM_WORKERS = NUM_CORES * NUM_SUBCORES
B_PER_W = BATCH // NUM_WORKERS

UNROLL = 16
N_BATCHES = B_PER_W // UNROLL

_mesh = plsc.VectorSubcoreMesh(
    core_axis_name="c", subcore_axis_name="s",
    num_cores=NUM_CORES, num_subcores=NUM_SUBCORES)


@functools.partial(
    pl.kernel,
    out_type=jax.ShapeDtypeStruct((BATCH, FEATURES), jnp.float32),
    mesh=_mesh,
    scratch_types=[
        pltpu.VMEM((B_PER_W,), jnp.int32),
        pltpu.VMEM((B_PER_W, FEATURES), jnp.float32),
        pltpu.SemaphoreType.DMA,
    ],
)
def _gather_kernel(table_hbm, idx_hbm, out_hbm, idx_v, rows_v, sem):
    wid = lax.axis_index("s") * NUM_CORES + lax.axis_index("c")
    base = wid * B_PER_W
    pltpu.sync_copy(idx_hbm.at[pl.ds(base, B_PER_W)], idx_v)

    def issue_batch(g, carry):
        vals = idx_v[pl.ds(g * UNROLL, UNROLL)]
        for j in range(UNROLL):
            row = vals[j]
            pltpu.async_copy(
                table_hbm.at[row], rows_v.at[g * UNROLL + j], sem)
        return carry

    lax.fori_loop(0, N_BATCHES, issue_batch, 0, unroll=False)
    pltpu.make_async_copy(
        table_hbm.at[pl.ds(0, B_PER_W)], rows_v, sem).wait()
    pltpu.sync_copy(rows_v, out_hbm.at[pl.ds(base, B_PER_W)])


def kernel(emb_vec, idx):
    return _gather_kernel(emb_vec, idx.astype(jnp.int32))
